# baseline (speedup 1.0000x reference)
import functools

import jax
import jax.numpy as jnp
from jax import lax
from jax.experimental import pallas as pl
from jax.experimental.pallas import tpu as pltpu, tpu_sc as plsc

N_NODES = 10000
N_EDGES = 160000
IN_FT = 256
OUT_FT = 256

NUM_CORES = 2
NUM_TILES = 16
HALF_FT = OUT_FT // NUM_CORES
CHUNK = 128
E_PAD = 16 * CHUNK * ((N_EDGES + 16 * CHUNK - 1) // (16 * CHUNK))
E_PER_TILE = E_PAD // NUM_TILES
N_CHUNKS = E_PER_TILE // CHUNK
ROWS_PER_TILE = 640
N_PAD = NUM_TILES * ROWS_PER_TILE
ROW_BLK = 128
N_ROW_BLKS = ROWS_PER_TILE // ROW_BLK
LANES = 16
VPR = HALF_FT // LANES


def _mm_body(seq_ref, w_ref, out_ref):
    out_ref[...] = lax.dot_general(
        seq_ref[...], w_ref[0],
        (((1,), (0,)), ((), ())),
        preferred_element_type=jnp.float32)


def _tc_matmul(seq, w2):
    n_blk = 1000
    n_grid = N_NODES // n_blk
    return pl.pallas_call(
        _mm_body,
        grid=(NUM_CORES, n_grid),
        in_specs=[
            pl.BlockSpec((n_blk, IN_FT), lambda p, i: (i, 0)),
            pl.BlockSpec((1, IN_FT, HALF_FT), lambda p, i: (p, 0, 0)),
        ],
        out_specs=pl.BlockSpec((n_blk, HALF_FT), lambda p, i: (p * n_grid + i, 0)),
        out_shape=jax.ShapeDtypeStruct((NUM_CORES * N_NODES, HALF_FT), jnp.float32),
    )(seq, w2)


def _sc_body(seqfts_hbm, edata_hbm, bias2_hbm, prelu_hbm,
             out_hbm, eb6, rbd, bias_v, prelu_v, acc_sh, gsem, ssem):
    c = lax.axis_index("c")
    s = lax.axis_index("s")

    zero16 = jnp.zeros((LANES,), jnp.float32)

    def zrow(r, carry):
        for j in range(VPR):
            rbd[r, pl.ds(j * LANES, LANES)] = zero16
        return carry

    lax.fori_loop(0, ROW_BLK, zrow, 0)
    r0 = s * ROWS_PER_TILE

    def zblk(b, carry):
        pltpu.sync_copy(rbd.at[pl.ds(0, ROW_BLK)],
                        acc_sh.at[pl.ds(r0 + b * ROW_BLK, ROW_BLK)])
        return carry

    lax.fori_loop(0, N_ROW_BLKS, zblk, 0)
    plsc.subcore_barrier()

    two16 = jnp.full((LANES,), 2, jnp.int32)
    ch0 = s * N_CHUNKS

    pltpu.async_copy(seqfts_hbm.at[pl.ds(0, CHUNK)],
                     rbd.at[pl.ds(CHUNK, CHUNK)], ssem)
    pltpu.sync_copy(edata_hbm.at[c, ch0], eb6.at[pl.ds(0, 3)])
    pltpu.async_copy(seqfts_hbm.at[eb6.at[0]], rbd.at[pl.ds(0, CHUNK)], gsem)
    last_chunk = E_PAD // CHUNK - 1

    def chunk_step(g, carry):
        m = g % 2
        e0 = m * 3
        rr0 = m * CHUNK
        en = (1 - m) * 3
        rrn = (1 - m) * CHUNK
        ebm = eb6.at[pl.ds(e0, 3)]
        rbm = rbd.at[pl.ds(rr0, CHUNK)]
        pltpu.make_async_copy(seqfts_hbm.at[pl.ds(0, CHUNK)], rbm, gsem).wait()
        def sq(q, cc):
            wvec = plsc.bitcast(eb6[e0 + 2, pl.ds(q * LANES, LANES)],
                                jnp.float32)
            base = rr0 + q * LANES
            for i in range(LANES):
                wspl = wvec.at[jnp.full((LANES,), i, jnp.int32)].get(
                    mode="promise_in_bounds")
                for j in range(VPR):
                    sl = pl.ds(j * LANES, LANES)
                    rbd[base + i, sl] = rbd[base + i, sl] * wspl
            return cc

        lax.fori_loop(0, CHUNK // LANES, sq, 0)
        pltpu.make_async_copy(seqfts_hbm.at[pl.ds(0, CHUNK)], rbm, ssem).wait()
        pltpu.async_copy(rbm, acc_sh.at[ebm.at[1]], ssem, add=True)
        kn = jnp.minimum(ch0 + g + 1, last_chunk)
        ebn = eb6.at[pl.ds(en, 3)]
        pltpu.sync_copy(edata_hbm.at[c, kn], ebn)
        pltpu.async_copy(seqfts_hbm.at[ebn.at[0]], rbd.at[pl.ds(rrn, CHUNK)],
                         gsem)
        return carry

    lax.fori_loop(0, N_CHUNKS, chunk_step, 0)
    pltpu.make_async_copy(seqfts_hbm.at[pl.ds(0, CHUNK)],
                          rbd.at[pl.ds(0, CHUNK)], gsem).wait()
    pltpu.make_async_copy(seqfts_hbm.at[pl.ds(0, CHUNK)],
                          rbd.at[pl.ds(0, CHUNK)], ssem).wait()
    plsc.subcore_barrier()

    pltpu.sync_copy(bias2_hbm.at[c], bias_v)
    pltpu.sync_copy(prelu_hbm, prelu_v)
    pv = prelu_v[...]
    bvs = [bias_v[pl.ds(j * LANES, LANES)] for j in range(VPR)]

    def eblk(b, carry):
        rbase = r0 + b * ROW_BLK
        pltpu.sync_copy(acc_sh.at[pl.ds(rbase, ROW_BLK)],
                        rbd.at[pl.ds(0, ROW_BLK)])

        def erow(r, cc):
            for j in range(VPR):
                sl = pl.ds(j * LANES, LANES)
                x = rbd[r, sl] + bvs[j]
                rbd[r, sl] = jnp.where(x > 0, x, x * pv)
            return cc

        lax.fori_loop(0, ROW_BLK, erow, 0)
        pltpu.sync_copy(
            rbd.at[pl.ds(0, ROW_BLK)],
            out_hbm.at[pl.ds(rbase, ROW_BLK), pl.ds(c * HALF_FT, HALF_FT)])
        return carry

    lax.fori_loop(0, N_ROW_BLKS, eblk, 0)


_sc_agg = functools.partial(
    pl.kernel,
    out_type=jax.ShapeDtypeStruct((N_PAD, OUT_FT), jnp.float32),
    mesh=plsc.VectorSubcoreMesh(core_axis_name="c", subcore_axis_name="s"),
    compiler_params=pltpu.CompilerParams(needs_layout_passes=False),
    scratch_types=[
        pltpu.VMEM((6, CHUNK), jnp.int32),
        pltpu.VMEM((2 * CHUNK, HALF_FT), jnp.float32),
        pltpu.VMEM((HALF_FT,), jnp.float32),
        pltpu.VMEM((LANES,), jnp.float32),
        pltpu.VMEM_SHARED((N_PAD, HALF_FT), jnp.float32),
        pltpu.SemaphoreType.DMA,
        pltpu.SemaphoreType.DMA,
    ],
)(_sc_body)


@jax.jit
def kernel(seq, edge_index, edge_weight, W, bias, prelu_a):
    src = edge_index[0].astype(jnp.int32)
    dst = edge_index[1].astype(jnp.int32)
    ew = edge_weight.astype(jnp.float32)

    pad = E_PAD - N_EDGES
    src = jnp.concatenate([src, jnp.zeros((pad,), jnp.int32)])
    dst = jnp.concatenate([dst, jnp.zeros((pad,), jnp.int32)])
    ew = jnp.concatenate([ew, jnp.zeros((pad,), jnp.float32)])
    srcs2 = jnp.stack([src, src + N_NODES])
    wbits = lax.bitcast_convert_type(ew, jnp.int32)
    n_chunks_tot = E_PAD // CHUNK
    edata = jnp.stack([
        srcs2,
        jnp.broadcast_to(dst, (NUM_CORES, E_PAD)),
        jnp.broadcast_to(wbits, (NUM_CORES, E_PAD)),
    ], axis=1)
    edata = edata.reshape(NUM_CORES, 3, n_chunks_tot, CHUNK)
    edata = edata.transpose(0, 2, 1, 3)

    w2 = W.T.reshape(IN_FT, NUM_CORES, HALF_FT).transpose(1, 0, 2)
    seqfts = _tc_matmul(seq, w2)

    bias2 = bias.reshape(NUM_CORES, HALF_FT)
    prelu16 = jnp.broadcast_to(prelu_a.reshape(1), (LANES,)).astype(jnp.float32)

    out = _sc_agg(seqfts, edata, bias2, prelu16)
    return out[:N_NODES]


# speedup vs baseline: 1.9212x; 1.9212x over previous
import functools

import jax
import jax.numpy as jnp
from jax import lax
from jax.experimental import pallas as pl
from jax.experimental.pallas import tpu as pltpu, tpu_sc as plsc

N_NODES = 10000
N_EDGES = 160000
IN_FT = 256
OUT_FT = 256

NUM_CORES = 2
NUM_TILES = 16
HALF_FT = OUT_FT // NUM_CORES
CHUNK = 128
E_PAD = 16 * CHUNK * ((N_EDGES + 16 * CHUNK - 1) // (16 * CHUNK))
E_PER_TILE = E_PAD // NUM_TILES
N_CHUNKS = E_PER_TILE // CHUNK
ROWS_PER_TILE = 640
N_PAD = NUM_TILES * ROWS_PER_TILE
ROW_BLK = 128
N_ROW_BLKS = ROWS_PER_TILE // ROW_BLK
LANES = 16
VPR = HALF_FT // LANES


def _mm_body(seq_ref, w_ref, out_ref):
    out_ref[...] = lax.dot_general(
        seq_ref[...], w_ref[0],
        (((1,), (0,)), ((), ())),
        preferred_element_type=jnp.float32)


def _tc_matmul(seq, w2):
    n_blk = 1000
    n_grid = N_NODES // n_blk
    return pl.pallas_call(
        _mm_body,
        grid=(NUM_CORES, n_grid),
        in_specs=[
            pl.BlockSpec((n_blk, IN_FT), lambda p, i: (i, 0)),
            pl.BlockSpec((1, IN_FT, HALF_FT), lambda p, i: (p, 0, 0)),
        ],
        out_specs=pl.BlockSpec((n_blk, HALF_FT), lambda p, i: (p * n_grid + i, 0)),
        out_shape=jax.ShapeDtypeStruct((NUM_CORES * N_NODES, HALF_FT), jnp.float32),
    )(seq, w2)


def _sc_body(seqfts_hbm, edata_hbm, bias2_hbm, prelu_hbm,
             out_hbm, eb6, rbd, bias_v, prelu_v, acc_sh, gsem, ssem):
    c = lax.axis_index("c")
    s = lax.axis_index("s")

    zero16 = jnp.zeros((LANES,), jnp.float32)

    def zrow(r, carry):
        for j in range(VPR):
            rbd[r, pl.ds(j * LANES, LANES)] = zero16
        return carry

    lax.fori_loop(0, ROW_BLK, zrow, 0)
    r0 = s * ROWS_PER_TILE

    def zblk(b, carry):
        pltpu.sync_copy(rbd.at[pl.ds(0, ROW_BLK)],
                        acc_sh.at[pl.ds(r0 + b * ROW_BLK, ROW_BLK)])
        return carry

    lax.fori_loop(0, N_ROW_BLKS, zblk, 0)
    plsc.subcore_barrier()

    two16 = jnp.full((LANES,), 2, jnp.int32)
    ch0 = s * N_CHUNKS

    pltpu.async_copy(seqfts_hbm.at[pl.ds(0, CHUNK)],
                     rbd.at[pl.ds(CHUNK, CHUNK)], ssem)
    pltpu.sync_copy(edata_hbm.at[c, ch0], eb6.at[pl.ds(0, 3)])
    pltpu.async_copy(seqfts_hbm.at[eb6.at[0]], rbd.at[pl.ds(0, CHUNK)], gsem)
    last_chunk = E_PAD // CHUNK - 1

    def chunk_step(g, carry):
        m = g % 2
        e0 = m * 3
        rr0 = m * CHUNK
        en = (1 - m) * 3
        rrn = (1 - m) * CHUNK
        ebm = eb6.at[pl.ds(e0, 3)]
        rbm = rbd.at[pl.ds(rr0, CHUNK)]
        ebn = eb6.at[pl.ds(en, 3)]
        pltpu.make_async_copy(seqfts_hbm.at[pl.ds(0, CHUNK)], rbm, ssem).wait()
        kn = jnp.minimum(ch0 + g + 1, last_chunk)
        pltpu.sync_copy(edata_hbm.at[c, kn], ebn)
        pltpu.make_async_copy(seqfts_hbm.at[pl.ds(0, CHUNK)], rbm, gsem).wait()
        pltpu.async_copy(seqfts_hbm.at[ebn.at[0]], rbd.at[pl.ds(rrn, CHUNK)],
                         gsem)
        w16 = jnp.broadcast_to(e0 + 2, (LANES,)).astype(jnp.int32)

        def srow(r, cc):
            widx = jnp.broadcast_to(r, (LANES,)).astype(jnp.int32)
            wspl = plsc.bitcast(
                plsc.load_gather(eb6, [w16, widx]), jnp.float32)
            for j in range(VPR):
                sl = pl.ds(j * LANES, LANES)
                rbd[rr0 + r, sl] = rbd[rr0 + r, sl] * wspl
            return cc

        lax.fori_loop(0, CHUNK, srow, 0)
        pltpu.async_copy(rbm, acc_sh.at[ebm.at[1]], ssem, add=True)
        return carry

    lax.fori_loop(0, N_CHUNKS, chunk_step, 0)
    pltpu.make_async_copy(seqfts_hbm.at[pl.ds(0, CHUNK)],
                          rbd.at[pl.ds(0, CHUNK)], gsem).wait()
    pltpu.make_async_copy(seqfts_hbm.at[pl.ds(0, CHUNK)],
                          rbd.at[pl.ds(0, CHUNK)], ssem).wait()
    plsc.subcore_barrier()

    pltpu.sync_copy(bias2_hbm.at[c], bias_v)
    pltpu.sync_copy(prelu_hbm, prelu_v)
    pv = prelu_v[...]
    bvs = [bias_v[pl.ds(j * LANES, LANES)] for j in range(VPR)]

    def eblk(b, carry):
        rbase = r0 + b * ROW_BLK
        pltpu.sync_copy(acc_sh.at[pl.ds(rbase, ROW_BLK)],
                        rbd.at[pl.ds(0, ROW_BLK)])

        def erow(r, cc):
            for j in range(VPR):
                sl = pl.ds(j * LANES, LANES)
                x = rbd[r, sl] + bvs[j]
                rbd[r, sl] = jnp.where(x > 0, x, x * pv)
            return cc

        lax.fori_loop(0, ROW_BLK, erow, 0)
        pltpu.sync_copy(
            rbd.at[pl.ds(0, ROW_BLK)],
            out_hbm.at[pl.ds(rbase, ROW_BLK), pl.ds(c * HALF_FT, HALF_FT)])
        return carry

    lax.fori_loop(0, N_ROW_BLKS, eblk, 0)


_sc_agg = functools.partial(
    pl.kernel,
    out_type=jax.ShapeDtypeStruct((N_PAD, OUT_FT), jnp.float32),
    mesh=plsc.VectorSubcoreMesh(core_axis_name="c", subcore_axis_name="s"),
    compiler_params=pltpu.CompilerParams(needs_layout_passes=False),
    scratch_types=[
        pltpu.VMEM((6, CHUNK), jnp.int32),
        pltpu.VMEM((2 * CHUNK, HALF_FT), jnp.float32),
        pltpu.VMEM((HALF_FT,), jnp.float32),
        pltpu.VMEM((LANES,), jnp.float32),
        pltpu.VMEM_SHARED((N_PAD, HALF_FT), jnp.float32),
        pltpu.SemaphoreType.DMA,
        pltpu.SemaphoreType.DMA,
    ],
)(_sc_body)


@jax.jit
def kernel(seq, edge_index, edge_weight, W, bias, prelu_a):
    src = edge_index[0].astype(jnp.int32)
    dst = edge_index[1].astype(jnp.int32)
    ew = edge_weight.astype(jnp.float32)

    pad = E_PAD - N_EDGES
    src = jnp.concatenate([src, jnp.zeros((pad,), jnp.int32)])
    dst = jnp.concatenate([dst, jnp.zeros((pad,), jnp.int32)])
    ew = jnp.concatenate([ew, jnp.zeros((pad,), jnp.float32)])
    srcs2 = jnp.stack([src, src + N_NODES])
    wbits = lax.bitcast_convert_type(ew, jnp.int32)
    n_chunks_tot = E_PAD // CHUNK
    edata = jnp.stack([
        srcs2,
        jnp.broadcast_to(dst, (NUM_CORES, E_PAD)),
        jnp.broadcast_to(wbits, (NUM_CORES, E_PAD)),
    ], axis=1)
    edata = edata.reshape(NUM_CORES, 3, n_chunks_tot, CHUNK)
    edata = edata.transpose(0, 2, 1, 3)

    w2 = W.T.reshape(IN_FT, NUM_CORES, HALF_FT).transpose(1, 0, 2)
    seqfts = _tc_matmul(seq, w2)

    bias2 = bias.reshape(NUM_CORES, HALF_FT)
    prelu16 = jnp.broadcast_to(prelu_a.reshape(1), (LANES,)).astype(jnp.float32)

    out = _sc_agg(seqfts, edata, bias2, prelu16)
    return out[:N_NODES]


# speedup vs baseline: 2.0955x; 1.0907x over previous
import functools

import jax
import jax.numpy as jnp
from jax import lax
from jax.experimental import pallas as pl
from jax.experimental.pallas import tpu as pltpu, tpu_sc as plsc

N_NODES = 10000
N_EDGES = 160000
IN_FT = 256
OUT_FT = 256

NUM_CORES = 2
NUM_TILES = 16
HALF_FT = OUT_FT // NUM_CORES
CHUNK = 128
E_PAD = 16 * CHUNK * ((N_EDGES + 16 * CHUNK - 1) // (16 * CHUNK))
E_PER_TILE = E_PAD // NUM_TILES
N_CHUNKS = E_PER_TILE // CHUNK
ROWS_PER_TILE = 640
N_PAD = NUM_TILES * ROWS_PER_TILE
ROW_BLK = 128
N_ROW_BLKS = ROWS_PER_TILE // ROW_BLK
LANES = 16
VPR = HALF_FT // LANES


def _mm_body(seq_ref, w_ref, out_ref):
    out_ref[...] = lax.dot_general(
        seq_ref[...], w_ref[0],
        (((1,), (0,)), ((), ())),
        preferred_element_type=jnp.float32)


def _tc_matmul(seq, w2):
    n_blk = 1000
    n_grid = N_NODES // n_blk
    return pl.pallas_call(
        _mm_body,
        grid=(NUM_CORES, n_grid),
        in_specs=[
            pl.BlockSpec((n_blk, IN_FT), lambda p, i: (i, 0)),
            pl.BlockSpec((1, IN_FT, HALF_FT), lambda p, i: (p, 0, 0)),
        ],
        out_specs=pl.BlockSpec((n_blk, HALF_FT), lambda p, i: (p * n_grid + i, 0)),
        out_shape=jax.ShapeDtypeStruct((NUM_CORES * N_NODES, HALF_FT), jnp.float32),
    )(seq, w2)


def _sc_body(seqfts_hbm, edata_hbm, bias2_hbm, prelu_hbm,
             out_hbm, eb6, rbd, bias_v, prelu_v, acc_sh, gsem, ssem, esem):
    c = lax.axis_index("c")
    s = lax.axis_index("s")

    zero16 = jnp.zeros((LANES,), jnp.float32)

    def zrow(r, carry):
        for j in range(VPR):
            rbd[r, pl.ds(j * LANES, LANES)] = zero16
        return carry

    lax.fori_loop(0, ROW_BLK, zrow, 0)
    r0 = s * ROWS_PER_TILE

    def zblk(b, carry):
        pltpu.sync_copy(rbd.at[pl.ds(0, ROW_BLK)],
                        acc_sh.at[pl.ds(r0 + b * ROW_BLK, ROW_BLK)])
        return carry

    lax.fori_loop(0, N_ROW_BLKS, zblk, 0)
    plsc.subcore_barrier()

    two16 = jnp.full((LANES,), 2, jnp.int32)
    ch0 = s * N_CHUNKS

    pltpu.async_copy(seqfts_hbm.at[pl.ds(0, CHUNK)],
                     rbd.at[pl.ds(CHUNK, CHUNK)], ssem)
    pltpu.sync_copy(edata_hbm.at[c, ch0], eb6.at[pl.ds(0, 3)])
    pltpu.async_copy(seqfts_hbm.at[eb6.at[0]], rbd.at[pl.ds(0, CHUNK)], gsem)
    pltpu.async_copy(edata_hbm.at[c, jnp.minimum(ch0 + 1, ch0 + N_CHUNKS - 1)],
                     eb6.at[pl.ds(3, 3)], esem)
    last_chunk = E_PAD // CHUNK - 1

    def chunk_step(g, carry):
        m = g % 2
        ea = (g % 3) * 3
        en = ((g + 1) % 3) * 3
        ef = ((g + 2) % 3) * 3
        rr0 = m * CHUNK
        rrn = (1 - m) * CHUNK
        ebm = eb6.at[pl.ds(ea, 3)]
        rbm = rbd.at[pl.ds(rr0, CHUNK)]
        ebn = eb6.at[pl.ds(en, 3)]
        pltpu.make_async_copy(seqfts_hbm.at[pl.ds(0, CHUNK)], rbm, ssem).wait()
        pltpu.make_async_copy(edata_hbm.at[c, 0], ebn, esem).wait()
        kf = jnp.minimum(ch0 + g + 2, last_chunk)
        pltpu.async_copy(edata_hbm.at[c, kf], eb6.at[pl.ds(ef, 3)], esem)
        pltpu.make_async_copy(seqfts_hbm.at[pl.ds(0, CHUNK)], rbm, gsem).wait()
        pltpu.async_copy(seqfts_hbm.at[ebn.at[0]], rbd.at[pl.ds(rrn, CHUNK)],
                         gsem)
        w16 = jnp.broadcast_to(ea + 2, (LANES,)).astype(jnp.int32)

        def srow(r, cc):
            widx = jnp.broadcast_to(r, (LANES,)).astype(jnp.int32)
            wspl = plsc.bitcast(
                plsc.load_gather(eb6, [w16, widx]), jnp.float32)
            for j in range(VPR):
                sl = pl.ds(j * LANES, LANES)
                rbd[rr0 + r, sl] = rbd[rr0 + r, sl] * wspl
            return cc

        lax.fori_loop(0, CHUNK, srow, 0)
        pltpu.async_copy(rbm, acc_sh.at[ebm.at[1]], ssem, add=True)
        return carry

    lax.fori_loop(0, N_CHUNKS, chunk_step, 0)
    pltpu.make_async_copy(edata_hbm.at[c, 0], eb6.at[pl.ds(0, 3)], esem).wait()
    pltpu.make_async_copy(seqfts_hbm.at[pl.ds(0, CHUNK)],
                          rbd.at[pl.ds(0, CHUNK)], gsem).wait()
    pltpu.make_async_copy(seqfts_hbm.at[pl.ds(0, CHUNK)],
                          rbd.at[pl.ds(0, CHUNK)], ssem).wait()
    plsc.subcore_barrier()

    pltpu.sync_copy(bias2_hbm.at[c], bias_v)
    pltpu.sync_copy(prelu_hbm, prelu_v)
    pv = prelu_v[...]
    bvs = [bias_v[pl.ds(j * LANES, LANES)] for j in range(VPR)]

    def eblk(b, carry):
        rbase = r0 + b * ROW_BLK
        pltpu.sync_copy(acc_sh.at[pl.ds(rbase, ROW_BLK)],
                        rbd.at[pl.ds(0, ROW_BLK)])

        def erow(r, cc):
            for j in range(VPR):
                sl = pl.ds(j * LANES, LANES)
                x = rbd[r, sl] + bvs[j]
                rbd[r, sl] = jnp.where(x > 0, x, x * pv)
            return cc

        lax.fori_loop(0, ROW_BLK, erow, 0)
        pltpu.sync_copy(
            rbd.at[pl.ds(0, ROW_BLK)],
            out_hbm.at[pl.ds(rbase, ROW_BLK), pl.ds(c * HALF_FT, HALF_FT)])
        return carry

    lax.fori_loop(0, N_ROW_BLKS, eblk, 0)


_sc_agg = functools.partial(
    pl.kernel,
    out_type=jax.ShapeDtypeStruct((N_PAD, OUT_FT), jnp.float32),
    mesh=plsc.VectorSubcoreMesh(core_axis_name="c", subcore_axis_name="s"),
    compiler_params=pltpu.CompilerParams(needs_layout_passes=False),
    scratch_types=[
        pltpu.VMEM((9, CHUNK), jnp.int32),
        pltpu.VMEM((2 * CHUNK, HALF_FT), jnp.float32),
        pltpu.VMEM((HALF_FT,), jnp.float32),
        pltpu.VMEM((LANES,), jnp.float32),
        pltpu.VMEM_SHARED((N_PAD, HALF_FT), jnp.float32),
        pltpu.SemaphoreType.DMA,
        pltpu.SemaphoreType.DMA,
        pltpu.SemaphoreType.DMA,
    ],
)(_sc_body)


@jax.jit
def kernel(seq, edge_index, edge_weight, W, bias, prelu_a):
    src = edge_index[0].astype(jnp.int32)
    dst = edge_index[1].astype(jnp.int32)
    ew = edge_weight.astype(jnp.float32)

    pad = E_PAD - N_EDGES
    src = jnp.concatenate([src, jnp.zeros((pad,), jnp.int32)])
    dst = jnp.concatenate([dst, jnp.zeros((pad,), jnp.int32)])
    ew = jnp.concatenate([ew, jnp.zeros((pad,), jnp.float32)])
    srcs2 = jnp.stack([src, src + N_NODES])
    wbits = lax.bitcast_convert_type(ew, jnp.int32)
    n_chunks_tot = E_PAD // CHUNK
    edata = jnp.stack([
        srcs2,
        jnp.broadcast_to(dst, (NUM_CORES, E_PAD)),
        jnp.broadcast_to(wbits, (NUM_CORES, E_PAD)),
    ], axis=1)
    edata = edata.reshape(NUM_CORES, 3, n_chunks_tot, CHUNK)
    edata = edata.transpose(0, 2, 1, 3)

    w2 = W.T.reshape(IN_FT, NUM_CORES, HALF_FT).transpose(1, 0, 2)
    seqfts = _tc_matmul(seq, w2)

    bias2 = bias.reshape(NUM_CORES, HALF_FT)
    prelu16 = jnp.broadcast_to(prelu_a.reshape(1), (LANES,)).astype(jnp.float32)

    out = _sc_agg(seqfts, edata, bias2, prelu16)
    return out[:N_NODES]
